# baseline (speedup 1.0000x reference)
import jax
import jax.numpy as jnp
from jax import lax
from jax.experimental import pallas as pl
from jax.experimental.pallas import tpu as pltpu
from jax.experimental.pallas import tpu_sc as plsc

N = 10000
D = 128
H = 64
E = 320000
N_PAD = 10240
ROWS_PER_TILE = 640
DUMMY = 10000
CHUNK = 128
NCHUNK = 160
E_PAD = 16 * NCHUNK * CHUNK
NBUF = 2
R_BLK = 640
GRID = N_PAD // R_BLK

_MESH = plsc.VectorSubcoreMesh(core_axis_name="c", subcore_axis_name="s")


def _mp_body(y_hbm, srcb_hbm, dstb_hbm, out_hbm, srcv, dstv, rows, acc,
             gsem0, gsem1):
    gsems = (gsem0, gsem1)
    c = lax.axis_index("c")
    s = lax.axis_index("s")
    r0 = s * ROWS_PER_TILE
    yc = y_hbm.at[c]
    pltpu.sync_copy(yc.at[pl.ds(r0, ROWS_PER_TILE)],
                    acc.at[pl.ds(r0, ROWS_PER_TILE)])
    pltpu.sync_copy(srcb_hbm.at[s], srcv)
    pltpu.sync_copy(dstb_hbm.at[s], dstv)
    plsc.subcore_barrier()

    for b in range(NBUF):
        pltpu.async_copy(yc.at[srcv.at[b]], rows.at[b], gsems[b])

    @pl.loop(0, NCHUNK, step=NBUF)
    def _(j):
        for b in range(NBUF):
            jj = j + b
            pltpu.make_async_copy(yc.at[srcv.at[b]], rows.at[b],
                                  gsems[b]).wait()
            pltpu.sync_copy(rows.at[b], acc.at[dstv.at[jj]], add=True)

            @pl.when(jj + NBUF < NCHUNK)
            def _():
                pltpu.async_copy(yc.at[srcv.at[jj + NBUF]], rows.at[b],
                                 gsems[b])

    plsc.subcore_barrier()
    pltpu.sync_copy(acc.at[pl.ds(r0, ROWS_PER_TILE)],
                    out_hbm.at[c].at[pl.ds(r0, ROWS_PER_TILE)])


_mp = pl.kernel(
    _mp_body,
    out_type=jax.ShapeDtypeStruct((2, N_PAD, H), jnp.float32),
    mesh=_MESH,
    compiler_params=pltpu.CompilerParams(use_tc_tiling_on_sc=False),
    scratch_types=[
        pltpu.VMEM((NCHUNK, CHUNK), jnp.int32),
        pltpu.VMEM((NCHUNK, CHUNK), jnp.int32),
        pltpu.VMEM((NBUF, CHUNK, H), jnp.float32),
        pltpu.VMEM_SHARED((N_PAD, H), jnp.float32),
        pltpu.SemaphoreType.DMA,
        pltpu.SemaphoreType.DMA,
    ],
)


DEG_CHUNKS = NCHUNK // 2


def _deg_body(ones_hbm, dstb_hbm, out_hbm, dstv, ones_v, acc,
              ssem0, ssem1, ssem2, ssem3):
    ssems = (ssem0, ssem1, ssem2, ssem3)
    c = lax.axis_index("c")
    s = lax.axis_index("s")
    r0 = s * ROWS_PER_TILE
    j0 = c * DEG_CHUNKS
    pltpu.sync_copy(ones_hbm.at[pl.ds(0, ROWS_PER_TILE)],
                    acc.at[pl.ds(r0, ROWS_PER_TILE)])
    pltpu.sync_copy(ones_hbm.at[pl.ds(0, CHUNK)], ones_v)
    pltpu.sync_copy(dstb_hbm.at[s], dstv)
    plsc.subcore_barrier()

    @pl.loop(j0, j0 + DEG_CHUNKS, step=4)
    def _(j):
        for b in range(4):
            pltpu.async_copy(ones_v, acc.at[dstv.at[j + b]], ssems[b],
                             add=True)
        for b in range(4):
            pltpu.make_async_copy(ones_v, acc.at[dstv.at[j + b]],
                                  ssems[b]).wait()

    plsc.subcore_barrier()
    pltpu.sync_copy(acc.at[pl.ds(r0, ROWS_PER_TILE)],
                    out_hbm.at[c].at[pl.ds(r0, ROWS_PER_TILE)])


_deg = pl.kernel(
    _deg_body,
    out_type=jax.ShapeDtypeStruct((2, N_PAD, H), jnp.float32),
    mesh=_MESH,
    compiler_params=pltpu.CompilerParams(use_tc_tiling_on_sc=False),
    scratch_types=[
        pltpu.VMEM((NCHUNK, CHUNK), jnp.int32),
        pltpu.VMEM((CHUNK, H), jnp.float32),
        pltpu.VMEM_SHARED((N_PAD, H), jnp.float32),
        pltpu.SemaphoreType.DMA,
        pltpu.SemaphoreType.DMA,
        pltpu.SemaphoreType.DMA,
        pltpu.SemaphoreType.DMA,
    ],
)


def _dinv_of(cd):
    deg = cd[0][:, 0:1] + cd[1][:, 0:1] - 1.0
    return lax.rsqrt(jnp.maximum(deg, 1.0))


def _mm1_body(x_ref, w_ref, cd_ref, y_ref):
    dinv = _dinv_of(cd_ref[...])
    res = dinv * jnp.dot(x_ref[...], w_ref[...],
                         preferred_element_type=jnp.float32,
                         precision=lax.Precision.HIGHEST)
    y_ref[0] = res[:, :H]
    y_ref[1] = res[:, H:]


def _mmh_body(s_ref, cd_ref, b_ref, w_ref, y_ref):
    dinv = _dinv_of(cd_ref[...])
    stot = jnp.concatenate([s_ref[0], s_ref[1]], axis=-1)
    h = jnp.maximum(dinv * stot + b_ref[...], 0.0)
    res = dinv * jnp.dot(h, w_ref[...],
                         preferred_element_type=jnp.float32,
                         precision=lax.Precision.HIGHEST)
    y_ref[0] = res[:, :H]
    y_ref[1] = res[:, H:]


def _fin_body(s_ref, cd_ref, b_ref, o_ref):
    dinv = _dinv_of(cd_ref[...])
    z = dinv * jnp.concatenate([s_ref[0], s_ref[1]], axis=-1) + b_ref[...]
    m = jnp.max(z, axis=1, keepdims=True)
    e = jnp.exp(z - m)
    o_ref[...] = z - (jnp.log(jnp.sum(e, axis=1, keepdims=True)) + m)


_row_spec = pl.BlockSpec((R_BLK, D), lambda i: (i, 0))
_half_spec = pl.BlockSpec((2, R_BLK, H), lambda i: (0, i, 0))
_cd_spec = pl.BlockSpec((2, R_BLK, H), lambda i: (0, i, 0))
_w_spec = pl.BlockSpec((D, D), lambda i: (0, 0))
_b_spec = pl.BlockSpec((1, D), lambda i: (0, 0))
_tc_params = pltpu.CompilerParams(
    dimension_semantics=("arbitrary",),
)

_mm1 = pl.pallas_call(
    _mm1_body,
    grid=(GRID,),
    in_specs=[_row_spec, _w_spec, _cd_spec],
    out_specs=_half_spec,
    out_shape=jax.ShapeDtypeStruct((2, N_PAD, H), jnp.float32),
    compiler_params=_tc_params,
)

_mmh = pl.pallas_call(
    _mmh_body,
    grid=(GRID,),
    in_specs=[_half_spec, _cd_spec, _b_spec, _w_spec],
    out_specs=_half_spec,
    out_shape=jax.ShapeDtypeStruct((2, N_PAD, H), jnp.float32),
    compiler_params=_tc_params,
)

_fin = pl.pallas_call(
    _fin_body,
    grid=(GRID,),
    in_specs=[_half_spec, _cd_spec, _b_spec],
    out_specs=_row_spec,
    out_shape=jax.ShapeDtypeStruct((N_PAD, D), jnp.float32),
    compiler_params=_tc_params,
)


def kernel(x, edge_index, W1, b1, Wh, bh, W2, b2):
    src = edge_index[0].astype(jnp.int32)
    dst = edge_index[1].astype(jnp.int32)
    srcb = jnp.concatenate(
        [src, jnp.zeros((E_PAD - E,), jnp.int32)]).reshape(16, NCHUNK, CHUNK)
    dstb = jnp.concatenate(
        [dst, jnp.full((E_PAD - E,), DUMMY, jnp.int32)]).reshape(16, NCHUNK, CHUNK)
    xp = jnp.pad(x, ((0, N_PAD - N), (0, 0)))

    cd = _deg(jnp.ones((N_PAD, H), jnp.float32), dstb)
    y1 = _mm1(xp, W1, cd)
    s1 = _mp(y1, srcb, dstb)
    y2 = _mmh(s1, cd, b1.reshape(1, D), Wh)
    s2 = _mp(y2, srcb, dstb)
    y3 = _mmh(s2, cd, bh.reshape(1, D), W2)
    s3 = _mp(y3, srcb, dstb)
    out = _fin(s3, cd, b2.reshape(1, D))
    return out[:N]


# speedup vs baseline: 1.7154x; 1.7154x over previous
import jax
import jax.numpy as jnp
from jax import lax
from jax.experimental import pallas as pl
from jax.experimental.pallas import tpu as pltpu
from jax.experimental.pallas import tpu_sc as plsc

N = 10000
D = 128
H = 64
E = 320000
N_PAD = 10240
ROWS_PER_TILE = 640
DUMMY = 10000
CHUNK = 128
NCHUNK = 158
E_PAD = 16 * NCHUNK * CHUNK
NBUF = 2
R_BLK = 640
GRID = N_PAD // R_BLK

_MESH = plsc.VectorSubcoreMesh(core_axis_name="c", subcore_axis_name="s")


def _mp_body(y_hbm, srcb_hbm, dstb_hbm, out_hbm, srcv, dstv, rows, acc,
             gsem0, gsem1, ssem0, ssem1):
    gsems = (gsem0, gsem1)
    ssems = (ssem0, ssem1)
    c = lax.axis_index("c")
    s = lax.axis_index("s")
    r0 = s * ROWS_PER_TILE
    yc = y_hbm.at[c]
    pltpu.sync_copy(yc.at[pl.ds(r0, ROWS_PER_TILE)],
                    acc.at[pl.ds(r0, ROWS_PER_TILE)])
    pltpu.sync_copy(srcb_hbm.at[s], srcv)
    pltpu.sync_copy(dstb_hbm.at[s], dstv)
    plsc.subcore_barrier()

    for b in range(NBUF):
        pltpu.async_copy(yc.at[srcv.at[b]], rows.at[b], gsems[b])

    @pl.loop(0, NCHUNK, step=NBUF)
    def _(j):
        for b in range(NBUF):
            pltpu.make_async_copy(yc.at[srcv.at[b]], rows.at[b],
                                  gsems[b]).wait()
            pltpu.async_copy(rows.at[b], acc.at[dstv.at[j + b]], ssems[b],
                             add=True)
        for b in range(NBUF):
            jj = j + b
            pltpu.make_async_copy(rows.at[b], acc.at[dstv.at[jj]],
                                  ssems[b]).wait()

            @pl.when(jj + NBUF < NCHUNK)
            def _():
                pltpu.async_copy(yc.at[srcv.at[jj + NBUF]], rows.at[b],
                                 gsems[b])

    plsc.subcore_barrier()
    pltpu.sync_copy(acc.at[pl.ds(r0, ROWS_PER_TILE)],
                    out_hbm.at[c].at[pl.ds(r0, ROWS_PER_TILE)])


_mp = pl.kernel(
    _mp_body,
    out_type=jax.ShapeDtypeStruct((2, N_PAD, H), jnp.float32),
    mesh=_MESH,
    compiler_params=pltpu.CompilerParams(use_tc_tiling_on_sc=False),
    scratch_types=[
        pltpu.VMEM((NCHUNK, CHUNK), jnp.int32),
        pltpu.VMEM((NCHUNK, CHUNK), jnp.int32),
        pltpu.VMEM((NBUF, CHUNK, H), jnp.float32),
        pltpu.VMEM_SHARED((N_PAD, H), jnp.float32),
        pltpu.SemaphoreType.DMA,
        pltpu.SemaphoreType.DMA,
        pltpu.SemaphoreType.DMA,
        pltpu.SemaphoreType.DMA,
    ],
)


DEG_CHUNKS = NCHUNK // 2


def _deg_body(ones_hbm, dstb_hbm, out_hbm, dstv, ones_v, acc,
              ssem0, ssem1, ssem2, ssem3):
    ssems = (ssem0, ssem1, ssem2, ssem3)
    c = lax.axis_index("c")
    s = lax.axis_index("s")
    r0 = s * ROWS_PER_TILE
    j0 = c * DEG_CHUNKS
    pltpu.sync_copy(ones_hbm.at[pl.ds(0, ROWS_PER_TILE)],
                    acc.at[pl.ds(r0, ROWS_PER_TILE)])
    pltpu.sync_copy(ones_hbm.at[pl.ds(0, CHUNK)], ones_v)
    pltpu.sync_copy(dstb_hbm.at[s], dstv)
    plsc.subcore_barrier()

    pltpu.sync_copy(ones_v, acc.at[dstv.at[j0]], add=True)

    @pl.loop(j0 + 1, j0 + DEG_CHUNKS, step=2)
    def _(j):
        for b in range(2):
            pltpu.async_copy(ones_v, acc.at[dstv.at[j + b]], ssems[b],
                             add=True)
        for b in range(2):
            pltpu.make_async_copy(ones_v, acc.at[dstv.at[j + b]],
                                  ssems[b]).wait()

    plsc.subcore_barrier()
    pltpu.sync_copy(acc.at[pl.ds(r0, ROWS_PER_TILE)],
                    out_hbm.at[c].at[pl.ds(r0, ROWS_PER_TILE)])


_deg = pl.kernel(
    _deg_body,
    out_type=jax.ShapeDtypeStruct((2, N_PAD, H), jnp.float32),
    mesh=_MESH,
    compiler_params=pltpu.CompilerParams(use_tc_tiling_on_sc=False),
    scratch_types=[
        pltpu.VMEM((NCHUNK, CHUNK), jnp.int32),
        pltpu.VMEM((CHUNK, H), jnp.float32),
        pltpu.VMEM_SHARED((N_PAD, H), jnp.float32),
        pltpu.SemaphoreType.DMA,
        pltpu.SemaphoreType.DMA,
        pltpu.SemaphoreType.DMA,
        pltpu.SemaphoreType.DMA,
    ],
)


def _dinv_of(cd):
    deg = cd[0][:, 0:1] + cd[1][:, 0:1] - 1.0
    return lax.rsqrt(jnp.maximum(deg, 1.0))


def _mm1_body(x_ref, w_ref, cd_ref, y_ref):
    dinv = _dinv_of(cd_ref[...])
    res = dinv * jnp.dot(x_ref[...], w_ref[...],
                         preferred_element_type=jnp.float32,
                         precision=lax.Precision.HIGHEST)
    y_ref[0] = res[:, :H]
    y_ref[1] = res[:, H:]


def _mmh_body(s_ref, cd_ref, b_ref, w_ref, y_ref):
    dinv = _dinv_of(cd_ref[...])
    stot = jnp.concatenate([s_ref[0], s_ref[1]], axis=-1)
    h = jnp.maximum(dinv * stot + b_ref[...], 0.0)
    res = dinv * jnp.dot(h, w_ref[...],
                         preferred_element_type=jnp.float32,
                         precision=lax.Precision.HIGHEST)
    y_ref[0] = res[:, :H]
    y_ref[1] = res[:, H:]


def _fin_body(s_ref, cd_ref, b_ref, o_ref):
    dinv = _dinv_of(cd_ref[...])
    z = dinv * jnp.concatenate([s_ref[0], s_ref[1]], axis=-1) + b_ref[...]
    m = jnp.max(z, axis=1, keepdims=True)
    e = jnp.exp(z - m)
    o_ref[...] = z - (jnp.log(jnp.sum(e, axis=1, keepdims=True)) + m)


_row_spec = pl.BlockSpec((R_BLK, D), lambda i: (i, 0))
_half_spec = pl.BlockSpec((2, R_BLK, H), lambda i: (0, i, 0))
_cd_spec = pl.BlockSpec((2, R_BLK, H), lambda i: (0, i, 0))
_w_spec = pl.BlockSpec((D, D), lambda i: (0, 0))
_b_spec = pl.BlockSpec((1, D), lambda i: (0, 0))
_tc_params = pltpu.CompilerParams(
    dimension_semantics=("arbitrary",),
)

_mm1 = pl.pallas_call(
    _mm1_body,
    grid=(GRID,),
    in_specs=[_row_spec, _w_spec, _cd_spec],
    out_specs=_half_spec,
    out_shape=jax.ShapeDtypeStruct((2, N_PAD, H), jnp.float32),
    compiler_params=_tc_params,
)

_mmh = pl.pallas_call(
    _mmh_body,
    grid=(GRID,),
    in_specs=[_half_spec, _cd_spec, _b_spec, _w_spec],
    out_specs=_half_spec,
    out_shape=jax.ShapeDtypeStruct((2, N_PAD, H), jnp.float32),
    compiler_params=_tc_params,
)

_fin = pl.pallas_call(
    _fin_body,
    grid=(GRID,),
    in_specs=[_half_spec, _cd_spec, _b_spec],
    out_specs=_row_spec,
    out_shape=jax.ShapeDtypeStruct((N_PAD, D), jnp.float32),
    compiler_params=_tc_params,
)


def kernel(x, edge_index, W1, b1, Wh, bh, W2, b2):
    src = edge_index[0].astype(jnp.int32)
    dst = edge_index[1].astype(jnp.int32)
    pad_i = jnp.arange(E_PAD - E, dtype=jnp.int32)
    srcb = jnp.concatenate(
        [src, pad_i % N]).reshape(16, NCHUNK, CHUNK)
    dstb = jnp.concatenate(
        [dst, DUMMY + pad_i % (N_PAD - N)]).reshape(16, NCHUNK, CHUNK)
    xp = jnp.pad(x, ((0, N_PAD - N), (0, 0)))

    cd = _deg(jnp.ones((N_PAD, H), jnp.float32), dstb)
    y1 = _mm1(xp, W1, cd)
    s1 = _mp(y1, srcb, dstb)
    y2 = _mmh(s1, cd, b1.reshape(1, D), Wh)
    s2 = _mp(y2, srcb, dstb)
    y3 = _mmh(s2, cd, bh.reshape(1, D), W2)
    s3 = _mp(y3, srcb, dstb)
    out = _fin(s3, cd, b2.reshape(1, D))
    return out[:N]


# speedup vs baseline: 2.1742x; 1.2674x over previous
import jax
import jax.numpy as jnp
from jax import lax
from jax.experimental import pallas as pl
from jax.experimental.pallas import tpu as pltpu
from jax.experimental.pallas import tpu_sc as plsc

N = 10000
D = 128
H = 64
E = 320000
N_PAD = 10240
ROWS_PER_TILE = 640
DUMMY = 10000
CHUNK = 128
NCHUNK = 160
E_PAD = 16 * NCHUNK * CHUNK
NBUF = 4
R_BLK = 640
GRID = N_PAD // R_BLK

_MESH = plsc.VectorSubcoreMesh(core_axis_name="c", subcore_axis_name="s")


def _mp_body(y_hbm, srcb_hbm, dstb_hbm, out_hbm, srcv, dstv, rows, acc,
             gsem0, gsem1, gsem2, gsem3, ssem0, ssem1, ssem2, ssem3):
    gsems = (gsem0, gsem1, gsem2, gsem3)
    ssems = (ssem0, ssem1, ssem2, ssem3)
    c = lax.axis_index("c")
    s = lax.axis_index("s")
    r0 = s * ROWS_PER_TILE
    yc = y_hbm.at[c]
    pltpu.sync_copy(yc.at[pl.ds(r0, ROWS_PER_TILE)],
                    acc.at[pl.ds(r0, ROWS_PER_TILE)])
    pltpu.sync_copy(srcb_hbm.at[s], srcv)
    pltpu.sync_copy(dstb_hbm.at[s], dstv)
    plsc.subcore_barrier()

    for b in range(NBUF):
        pltpu.async_copy(yc.at[srcv.at[b]], rows.at[b], gsems[b])

    @pl.loop(0, NCHUNK, step=NBUF)
    def _(j):
        for b in range(NBUF):
            pltpu.make_async_copy(yc.at[srcv.at[b]], rows.at[b],
                                  gsems[b]).wait()
            pltpu.async_copy(rows.at[b], acc.at[dstv.at[j + b]], ssems[b],
                             add=True)
        for b in range(NBUF):
            jj = j + b
            pltpu.make_async_copy(rows.at[b], acc.at[dstv.at[jj]],
                                  ssems[b]).wait()

            @pl.when(jj + NBUF < NCHUNK)
            def _():
                pltpu.async_copy(yc.at[srcv.at[jj + NBUF]], rows.at[b],
                                 gsems[b])

    plsc.subcore_barrier()
    pltpu.sync_copy(acc.at[pl.ds(r0, ROWS_PER_TILE)],
                    out_hbm.at[c].at[pl.ds(r0, ROWS_PER_TILE)])


_mp = pl.kernel(
    _mp_body,
    out_type=jax.ShapeDtypeStruct((2, N_PAD, H), jnp.float32),
    mesh=_MESH,
    compiler_params=pltpu.CompilerParams(use_tc_tiling_on_sc=False),
    scratch_types=[
        pltpu.VMEM((NCHUNK, CHUNK), jnp.int32),
        pltpu.VMEM((NCHUNK, CHUNK), jnp.int32),
        pltpu.VMEM((NBUF, CHUNK, H), jnp.float32),
        pltpu.VMEM_SHARED((N_PAD, H), jnp.float32),
        pltpu.SemaphoreType.DMA,
        pltpu.SemaphoreType.DMA,
        pltpu.SemaphoreType.DMA,
        pltpu.SemaphoreType.DMA,
        pltpu.SemaphoreType.DMA,
        pltpu.SemaphoreType.DMA,
        pltpu.SemaphoreType.DMA,
        pltpu.SemaphoreType.DMA,
    ],
)


DEG_CHUNKS = NCHUNK // 2


def _deg_body(ones_hbm, dstb_hbm, out_hbm, dstv, ones_v, acc,
              ssem0, ssem1, ssem2, ssem3):
    ssems = (ssem0, ssem1, ssem2, ssem3)
    c = lax.axis_index("c")
    s = lax.axis_index("s")
    r0 = s * ROWS_PER_TILE
    j0 = c * DEG_CHUNKS
    pltpu.sync_copy(ones_hbm.at[pl.ds(0, ROWS_PER_TILE)],
                    acc.at[pl.ds(r0, ROWS_PER_TILE)])
    pltpu.sync_copy(ones_hbm.at[pl.ds(0, CHUNK)], ones_v)
    pltpu.sync_copy(dstb_hbm.at[s], dstv)
    plsc.subcore_barrier()

    @pl.loop(j0, j0 + DEG_CHUNKS, step=4)
    def _(j):
        for b in range(4):
            pltpu.async_copy(ones_v, acc.at[dstv.at[j + b]], ssems[b],
                             add=True)
        for b in range(4):
            pltpu.make_async_copy(ones_v, acc.at[dstv.at[j + b]],
                                  ssems[b]).wait()

    plsc.subcore_barrier()
    pltpu.sync_copy(acc.at[pl.ds(r0, ROWS_PER_TILE)],
                    out_hbm.at[c].at[pl.ds(r0, ROWS_PER_TILE)])


_deg = pl.kernel(
    _deg_body,
    out_type=jax.ShapeDtypeStruct((2, N_PAD, H), jnp.float32),
    mesh=_MESH,
    compiler_params=pltpu.CompilerParams(use_tc_tiling_on_sc=False),
    scratch_types=[
        pltpu.VMEM((NCHUNK, CHUNK), jnp.int32),
        pltpu.VMEM((CHUNK, H), jnp.float32),
        pltpu.VMEM_SHARED((N_PAD, H), jnp.float32),
        pltpu.SemaphoreType.DMA,
        pltpu.SemaphoreType.DMA,
        pltpu.SemaphoreType.DMA,
        pltpu.SemaphoreType.DMA,
    ],
)


def _dinv_of(cd):
    deg = cd[0][:, 0:1] + cd[1][:, 0:1] - 1.0
    return lax.rsqrt(jnp.maximum(deg, 1.0))


def _mm1_body(x_ref, w_ref, cd_ref, y_ref):
    dinv = _dinv_of(cd_ref[...])
    res = dinv * jnp.dot(x_ref[...], w_ref[...],
                         preferred_element_type=jnp.float32,
                         precision=lax.Precision.HIGHEST)
    y_ref[0] = res[:, :H]
    y_ref[1] = res[:, H:]


def _mmh_body(s_ref, cd_ref, b_ref, w_ref, y_ref):
    dinv = _dinv_of(cd_ref[...])
    stot = jnp.concatenate([s_ref[0], s_ref[1]], axis=-1)
    h = jnp.maximum(dinv * stot + b_ref[...], 0.0)
    res = dinv * jnp.dot(h, w_ref[...],
                         preferred_element_type=jnp.float32,
                         precision=lax.Precision.HIGHEST)
    y_ref[0] = res[:, :H]
    y_ref[1] = res[:, H:]


def _fin_body(s_ref, cd_ref, b_ref, o_ref):
    dinv = _dinv_of(cd_ref[...])
    z = dinv * jnp.concatenate([s_ref[0], s_ref[1]], axis=-1) + b_ref[...]
    m = jnp.max(z, axis=1, keepdims=True)
    e = jnp.exp(z - m)
    o_ref[...] = z - (jnp.log(jnp.sum(e, axis=1, keepdims=True)) + m)


_row_spec = pl.BlockSpec((R_BLK, D), lambda i: (i, 0))
_half_spec = pl.BlockSpec((2, R_BLK, H), lambda i: (0, i, 0))
_cd_spec = pl.BlockSpec((2, R_BLK, H), lambda i: (0, i, 0))
_w_spec = pl.BlockSpec((D, D), lambda i: (0, 0))
_b_spec = pl.BlockSpec((1, D), lambda i: (0, 0))
_tc_params = pltpu.CompilerParams(
    dimension_semantics=("arbitrary",),
)

_mm1 = pl.pallas_call(
    _mm1_body,
    grid=(GRID,),
    in_specs=[_row_spec, _w_spec, _cd_spec],
    out_specs=_half_spec,
    out_shape=jax.ShapeDtypeStruct((2, N_PAD, H), jnp.float32),
    compiler_params=_tc_params,
)

_mmh = pl.pallas_call(
    _mmh_body,
    grid=(GRID,),
    in_specs=[_half_spec, _cd_spec, _b_spec, _w_spec],
    out_specs=_half_spec,
    out_shape=jax.ShapeDtypeStruct((2, N_PAD, H), jnp.float32),
    compiler_params=_tc_params,
)

_fin = pl.pallas_call(
    _fin_body,
    grid=(GRID,),
    in_specs=[_half_spec, _cd_spec, _b_spec],
    out_specs=_row_spec,
    out_shape=jax.ShapeDtypeStruct((N_PAD, D), jnp.float32),
    compiler_params=_tc_params,
)


def kernel(x, edge_index, W1, b1, Wh, bh, W2, b2):
    src = edge_index[0].astype(jnp.int32)
    dst = edge_index[1].astype(jnp.int32)
    pad_i = jnp.arange(E_PAD - E, dtype=jnp.int32)
    srcb = jnp.concatenate(
        [src, pad_i % N]).reshape(16, NCHUNK, CHUNK)
    dstb = jnp.concatenate(
        [dst, DUMMY + pad_i % (N_PAD - N)]).reshape(16, NCHUNK, CHUNK)
    xp = jnp.pad(x, ((0, N_PAD - N), (0, 0)))

    cd = _deg(jnp.ones((N_PAD, H), jnp.float32), dstb)
    y1 = _mm1(xp, W1, cd)
    s1 = _mp(y1, srcb, dstb)
    y2 = _mmh(s1, cd, b1.reshape(1, D), Wh)
    s2 = _mp(y2, srcb, dstb)
    y3 = _mmh(s2, cd, bh.reshape(1, D), W2)
    s3 = _mp(y3, srcb, dstb)
    out = _fin(s3, cd, b2.reshape(1, D))
    return out[:N]
